# baseline (speedup 1.0000x reference)
import jax
import jax.numpy as jnp
from jax.experimental import pallas as pl
from jax.experimental.pallas import tpu as pltpu
from jax.experimental.pallas import tpu_sc as plsc

B, S, H = 1, 2048, 768
E, K, I = 16, 2, 256
T = B * S
MT = 128
M_PAD = 2 * T + E * MT
NT = M_PAD // MT
HH = H // 2
W = 128


def _silu(v):
    return v * jax.nn.sigmoid(v)


def _router_kernel(x_ref, wgate_ref, pos_ref, te_ref, w1_ref, w2_ref):
    logits = jnp.dot(x_ref[...], wgate_ref[...], preferred_element_type=jnp.float32)
    iota_e = jax.lax.broadcasted_iota(jnp.int32, (T, E), 1)
    a1 = jnp.argmax(logits, axis=-1)
    hot1 = iota_e == a1[:, None]
    m1 = jnp.max(logits, axis=-1, keepdims=True)
    masked = jnp.where(hot1, -jnp.inf, logits)
    a2 = jnp.argmax(masked, axis=-1)
    hot2 = iota_e == a2[:, None]
    m2 = jnp.max(masked, axis=-1, keepdims=True)
    e2 = jnp.exp(m2 - m1)
    w1_ref[...] = 1.0 / (1.0 + e2)
    w2_ref[...] = e2 / (1.0 + e2)

    oh = jnp.concatenate([hot1, hot2], axis=0).astype(jnp.float32)
    ri = jax.lax.broadcasted_iota(jnp.int32, (MT, MT), 0)
    ci = jax.lax.broadcasted_iota(jnp.int32, (MT, MT), 1)
    ltri = (ci < ri).astype(jnp.float32)
    parts = []
    running = jnp.zeros((1, E), jnp.float32)
    for b in range(2 * T // MT):
        blk = oh[b * MT:(b + 1) * MT]
        within = jnp.dot(ltri, blk, preferred_element_type=jnp.float32)
        parts.append(within + running)
        running = running + jnp.sum(blk, axis=0, keepdims=True)
    excl = jnp.concatenate(parts, axis=0)
    rank = jnp.sum(excl * oh, axis=-1)
    seg = jnp.ceil(running / float(MT)) * float(MT)
    segm = jnp.broadcast_to(seg, (E, E))
    je = jax.lax.broadcasted_iota(jnp.int32, (E, E), 1)
    ie = jax.lax.broadcasted_iota(jnp.int32, (E, E), 0)
    base = jnp.sum(jnp.where(je < ie, segm, 0.0), axis=-1)
    base_slot = jnp.sum(oh * base[None, :], axis=-1)
    p = (base_slot + rank).astype(jnp.int32).reshape(2 * T, 1)
    pos_ref[...] = jnp.concatenate([2 * p, 2 * p + 1], axis=1)
    tile_start = jax.lax.broadcasted_iota(jnp.int32, (NT, E), 0) * MT
    base_i = jnp.broadcast_to(base[None, :], (NT, E)).astype(jnp.int32)
    te = jnp.sum((base_i <= tile_start).astype(jnp.int32), axis=-1) - 1
    te_ref[...] = te.reshape(NT, 1)


def _gmm_kernel(te_ref, xs_ref, wg_ref, wu_ref, wd_ref, ys_ref):
    xt = xs_ref[...]
    g = jnp.dot(xt, wg_ref[0], preferred_element_type=jnp.float32)
    u = jnp.dot(xt, wu_ref[0], preferred_element_type=jnp.float32)
    ys_ref[...] = jnp.dot(_silu(g) * u, wd_ref[0], preferred_element_type=jnp.float32)


def _shared_kernel(x_ref, sg_ref, su_ref, sd_ref, o_ref):
    x = x_ref[...]
    gs = jnp.dot(x, sg_ref[...], preferred_element_type=jnp.float32)
    us = jnp.dot(x, su_ref[...], preferred_element_type=jnp.float32)
    o_ref[...] = jnp.dot(_silu(gs) * us, sd_ref[...], preferred_element_type=jnp.float32)


def _combine_kernel(sh_ref, g_ref, w1_ref, w2_ref, o_ref):
    o_ref[...] = (sh_ref[...]
                  + w1_ref[...] * g_ref[0:T, :]
                  + w2_ref[...] * g_ref[T:2 * T, :])


def _vec_mesh():
    return plsc.VectorSubcoreMesh(core_axis_name="core", subcore_axis_name="subcore")


def _sc_scatter(xf_hbm, pos_hbm, xs_hbm):
    def body(x_vmem, i_vmem):
        pltpu.sync_copy(x_vmem, xs_hbm.at[i_vmem.at[0]])

    pltpu.emit_pipeline(
        body,
        grid=(4 * T // W,),
        in_specs=[
            pl.BlockSpec((W, HH), lambda i: (i % (2 * T // W), 0)),
            pl.BlockSpec((1, W), lambda i: (0, i)),
        ],
        out_specs=[],
        core_axis_name=("core", "subcore"),
        dimension_semantics=(pltpu.PARALLEL,),
    )(xf_hbm, pos_hbm)


def _sc_gather(ys_hbm, pos_hbm, g_hbm):
    def body(i_vmem, o_vmem):
        pltpu.sync_copy(ys_hbm.at[i_vmem.at[0]], o_vmem)

    pltpu.emit_pipeline(
        body,
        grid=(4 * T // W,),
        in_specs=[pl.BlockSpec((1, W), lambda i: (0, i))],
        out_specs=[pl.BlockSpec((W, HH), lambda i: (i, 0))],
        core_axis_name=("core", "subcore"),
        dimension_semantics=(pltpu.PARALLEL,),
    )(pos_hbm, g_hbm)


def kernel(x, w_gate, wg, wu, wd, sg, su, sd):
    xf = x.reshape(T, H)

    pos, te, w1, w2 = pl.pallas_call(
        _router_kernel,
        out_shape=(
            jax.ShapeDtypeStruct((2 * T, 2), jnp.int32),
            jax.ShapeDtypeStruct((NT, 1), jnp.int32),
            jax.ShapeDtypeStruct((T, 1), jnp.float32),
            jax.ShapeDtypeStruct((T, 1), jnp.float32),
        ),
    )(xf, w_gate)

    pos_row = pos.reshape(1, 4 * T)
    te1d = te.reshape(NT)

    sc_scatter = pl.kernel(
        _sc_scatter,
        out_type=jax.ShapeDtypeStruct((2 * M_PAD, HH), jnp.float32),
        mesh=_vec_mesh(),
    )
    xs = sc_scatter(xf.reshape(2 * T, HH), pos_row).reshape(M_PAD, H)

    ys = pl.pallas_call(
        _gmm_kernel,
        grid_spec=pltpu.PrefetchScalarGridSpec(
            num_scalar_prefetch=1,
            grid=(NT,),
            in_specs=[
                pl.BlockSpec((MT, H), lambda i, te: (i, 0)),
                pl.BlockSpec((1, H, I), lambda i, te: (te[i], 0, 0)),
                pl.BlockSpec((1, H, I), lambda i, te: (te[i], 0, 0)),
                pl.BlockSpec((1, I, H), lambda i, te: (te[i], 0, 0)),
            ],
            out_specs=pl.BlockSpec((MT, H), lambda i, te: (i, 0)),
        ),
        out_shape=jax.ShapeDtypeStruct((M_PAD, H), jnp.float32),
        compiler_params=pltpu.CompilerParams(dimension_semantics=("arbitrary",)),
    )(te1d, xs, wg, wu, wd)

    sc_gather = pl.kernel(
        _sc_gather,
        out_type=jax.ShapeDtypeStruct((4 * T, HH), jnp.float32),
        mesh=_vec_mesh(),
    )
    g = sc_gather(ys.reshape(2 * M_PAD, HH), pos_row).reshape(2 * T, H)

    shared = pl.pallas_call(
        _shared_kernel,
        out_shape=jax.ShapeDtypeStruct((T, H), jnp.float32),
    )(xf, sg, su, sd)

    y = pl.pallas_call(
        _combine_kernel,
        out_shape=jax.ShapeDtypeStruct((T, H), jnp.float32),
    )(shared, g, w1, w2)

    return y.reshape(B, S, H)


# speedup vs baseline: 1.5230x; 1.5230x over previous
import jax
import jax.numpy as jnp
from jax.experimental import pallas as pl
from jax.experimental.pallas import tpu as pltpu
from jax.experimental.pallas import tpu_sc as plsc

B, S, H = 1, 2048, 768
E, K, I = 16, 2, 256
T = B * S
MT = 128
M_PAD = 2 * T + E * MT
NT = M_PAD // MT
HH = H // 2
W = 128


def _silu(v):
    return v * jax.nn.sigmoid(v)


def _router_kernel(x_ref, wgate_ref, pos_ref, te_ref, w1_ref, w2_ref):
    logits = jnp.dot(x_ref[...], wgate_ref[...], preferred_element_type=jnp.float32)
    iota_e = jax.lax.broadcasted_iota(jnp.int32, (T, E), 1)
    a1 = jnp.argmax(logits, axis=-1)
    hot1 = iota_e == a1[:, None]
    m1 = jnp.max(logits, axis=-1, keepdims=True)
    masked = jnp.where(hot1, -jnp.inf, logits)
    a2 = jnp.argmax(masked, axis=-1)
    hot2 = iota_e == a2[:, None]
    m2 = jnp.max(masked, axis=-1, keepdims=True)
    e2 = jnp.exp(m2 - m1)
    w1_ref[...] = 1.0 / (1.0 + e2)
    w2_ref[...] = e2 / (1.0 + e2)

    oh = jnp.concatenate([hot1, hot2], axis=0).astype(jnp.float32)
    ri = jax.lax.broadcasted_iota(jnp.int32, (MT, MT), 0)
    ci = jax.lax.broadcasted_iota(jnp.int32, (MT, MT), 1)
    ltri = (ci < ri).astype(jnp.float32)
    parts = []
    running = jnp.zeros((1, E), jnp.float32)
    for b in range(2 * T // MT):
        blk = oh[b * MT:(b + 1) * MT]
        within = jnp.dot(ltri, blk, preferred_element_type=jnp.float32)
        parts.append(within + running)
        running = running + jnp.sum(blk, axis=0, keepdims=True)
    excl = jnp.concatenate(parts, axis=0)
    rank = jnp.sum(excl * oh, axis=-1)
    seg = jnp.ceil(running / float(MT)) * float(MT)
    segm = jnp.broadcast_to(seg, (E, E))
    je = jax.lax.broadcasted_iota(jnp.int32, (E, E), 1)
    ie = jax.lax.broadcasted_iota(jnp.int32, (E, E), 0)
    base = jnp.sum(jnp.where(je < ie, segm, 0.0), axis=-1)
    base_slot = jnp.sum(oh * base[None, :], axis=-1)
    pos_ref[...] = (base_slot + rank).astype(jnp.int32).reshape(1, 2 * T)
    tile_start = jax.lax.broadcasted_iota(jnp.int32, (NT, E), 0) * MT
    base_i = jnp.broadcast_to(base[None, :], (NT, E)).astype(jnp.int32)
    te_ref[...] = jnp.sum((base_i <= tile_start).astype(jnp.int32),
                          axis=-1, keepdims=True) - 1


def _gmm_kernel(te_ref, xl_ref, xr_ref, wg_ref, wu_ref, wd_ref, yl_ref, yr_ref):
    e = te_ref[pl.program_id(0), 0]
    xl = xl_ref[...]
    xr = xr_ref[...]
    wge = wg_ref[e]
    wue = wu_ref[e]
    wde = wd_ref[e]
    g = (jnp.dot(xl, wge[:HH], preferred_element_type=jnp.float32)
         + jnp.dot(xr, wge[HH:], preferred_element_type=jnp.float32))
    u = (jnp.dot(xl, wue[:HH], preferred_element_type=jnp.float32)
         + jnp.dot(xr, wue[HH:], preferred_element_type=jnp.float32))
    h = _silu(g) * u
    yl_ref[...] = jnp.dot(h, wde[:, :HH], preferred_element_type=jnp.float32)
    yr_ref[...] = jnp.dot(h, wde[:, HH:], preferred_element_type=jnp.float32)


def _shared_kernel(x_ref, sg_ref, su_ref, sd_ref, o_ref):
    x = x_ref[...]
    gs = jnp.dot(x, sg_ref[...], preferred_element_type=jnp.float32)
    us = jnp.dot(x, su_ref[...], preferred_element_type=jnp.float32)
    o_ref[...] = jnp.dot(_silu(gs) * us, sd_ref[...], preferred_element_type=jnp.float32)


def _combine_kernel(sh_ref, gl_ref, gr_ref, w1_ref, w2_ref, o_ref):
    w1 = w1_ref[...]
    w2 = w2_ref[...]
    o_ref[:, :HH] = (sh_ref[:, :HH]
                     + w1 * gl_ref[0:T, :] + w2 * gl_ref[T:2 * T, :])
    o_ref[:, HH:] = (sh_ref[:, HH:]
                     + w1 * gr_ref[0:T, :] + w2 * gr_ref[T:2 * T, :])


def _vec_mesh():
    return plsc.VectorSubcoreMesh(core_axis_name="core", subcore_axis_name="subcore")


def _sc_scatter(xf_hbm, pos_hbm, xsl_hbm, xsr_hbm):
    def body_l(x_vmem, i_vmem):
        pltpu.sync_copy(x_vmem, xsl_hbm.at[i_vmem.at[0]])

    def body_r(x_vmem, i_vmem):
        pltpu.sync_copy(x_vmem, xsr_hbm.at[i_vmem.at[0]])

    for body, col in ((body_l, 0), (body_r, 1)):
        pltpu.emit_pipeline(
            body,
            grid=(2 * T // W,),
            in_specs=[
                pl.BlockSpec((W, HH), lambda i, c=col: (i % (T // W), c)),
                pl.BlockSpec((1, W), lambda i: (0, i)),
            ],
            out_specs=[],
            core_axis_name=("core", "subcore"),
            dimension_semantics=(pltpu.PARALLEL,),
        )(xf_hbm, pos_hbm)


def _sc_gather(ysl_hbm, ysr_hbm, pos_hbm, gl_hbm, gr_hbm):
    def body_l(i_vmem, o_vmem):
        pltpu.sync_copy(ysl_hbm.at[i_vmem.at[0]], o_vmem)

    def body_r(i_vmem, o_vmem):
        pltpu.sync_copy(ysr_hbm.at[i_vmem.at[0]], o_vmem)

    for body, out in ((body_l, gl_hbm), (body_r, gr_hbm)):
        pltpu.emit_pipeline(
            body,
            grid=(2 * T // W,),
            in_specs=[pl.BlockSpec((1, W), lambda i: (0, i))],
            out_specs=[pl.BlockSpec((W, HH), lambda i: (i, 0))],
            core_axis_name=("core", "subcore"),
            dimension_semantics=(pltpu.PARALLEL,),
        )(pos_hbm, out)


def kernel(x, w_gate, wg, wu, wd, sg, su, sd):
    xf = x.reshape(T, H)

    pos, te, w1, w2 = pl.pallas_call(
        _router_kernel,
        out_shape=(
            jax.ShapeDtypeStruct((1, 2 * T), jnp.int32),
            jax.ShapeDtypeStruct((NT, 1), jnp.int32),
            jax.ShapeDtypeStruct((T, 1), jnp.float32),
            jax.ShapeDtypeStruct((T, 1), jnp.float32),
        ),
    )(xf, w_gate)

    sc_scatter = pl.kernel(
        _sc_scatter,
        out_type=(jax.ShapeDtypeStruct((M_PAD, HH), jnp.float32),
                  jax.ShapeDtypeStruct((M_PAD, HH), jnp.float32)),
        mesh=_vec_mesh(),
    )
    xsl, xsr = sc_scatter(xf, pos)

    ysl, ysr = pl.pallas_call(
        _gmm_kernel,
        grid=(NT,),
        in_specs=[
            pl.BlockSpec(memory_space=pltpu.SMEM),
            pl.BlockSpec((MT, HH), lambda i: (i, 0)),
            pl.BlockSpec((MT, HH), lambda i: (i, 0)),
            pl.BlockSpec((E, H, I), lambda i: (0, 0, 0)),
            pl.BlockSpec((E, H, I), lambda i: (0, 0, 0)),
            pl.BlockSpec((E, I, H), lambda i: (0, 0, 0)),
        ],
        out_specs=(pl.BlockSpec((MT, HH), lambda i: (i, 0)),
                   pl.BlockSpec((MT, HH), lambda i: (i, 0))),
        out_shape=(jax.ShapeDtypeStruct((M_PAD, HH), jnp.float32),
                   jax.ShapeDtypeStruct((M_PAD, HH), jnp.float32)),
        compiler_params=pltpu.CompilerParams(dimension_semantics=("arbitrary",)),
    )(te, xsl, xsr, wg, wu, wd)

    sc_gather = pl.kernel(
        _sc_gather,
        out_type=(jax.ShapeDtypeStruct((2 * T, HH), jnp.float32),
                  jax.ShapeDtypeStruct((2 * T, HH), jnp.float32)),
        mesh=_vec_mesh(),
    )
    gl, gr = sc_gather(ysl, ysr, pos)

    shared = pl.pallas_call(
        _shared_kernel,
        out_shape=jax.ShapeDtypeStruct((T, H), jnp.float32),
    )(xf, sg, su, sd)

    y = pl.pallas_call(
        _combine_kernel,
        out_shape=jax.ShapeDtypeStruct((T, H), jnp.float32),
    )(shared, gl, gr, w1, w2)

    return y.reshape(B, S, H)


# speedup vs baseline: 1.5962x; 1.0480x over previous
import jax
import jax.numpy as jnp
from jax.experimental import pallas as pl
from jax.experimental.pallas import tpu as pltpu
from jax.experimental.pallas import tpu_sc as plsc

B, S, H = 1, 2048, 768
E, K, I = 16, 2, 256
T = B * S
MT = 256
M_PAD = 2 * T + E * MT
NT = M_PAD // MT
HH = H // 2
W = 128


def _silu(v):
    return v * jax.nn.sigmoid(v)


def _router_kernel(x_ref, wgate_ref, pos_ref, te_ref, w1_ref, w2_ref):
    logits = jnp.dot(x_ref[...], wgate_ref[...], preferred_element_type=jnp.float32)
    iota_e = jax.lax.broadcasted_iota(jnp.int32, (T, E), 1)
    a1 = jnp.argmax(logits, axis=-1)
    hot1 = iota_e == a1[:, None]
    m1 = jnp.max(logits, axis=-1, keepdims=True)
    masked = jnp.where(hot1, -jnp.inf, logits)
    a2 = jnp.argmax(masked, axis=-1)
    hot2 = iota_e == a2[:, None]
    m2 = jnp.max(masked, axis=-1, keepdims=True)
    e2 = jnp.exp(m2 - m1)
    w1_ref[...] = 1.0 / (1.0 + e2)
    w2_ref[...] = e2 / (1.0 + e2)

    oh = jnp.concatenate([hot1, hot2], axis=0).astype(jnp.float32)
    ri = jax.lax.broadcasted_iota(jnp.int32, (MT, MT), 0)
    ci = jax.lax.broadcasted_iota(jnp.int32, (MT, MT), 1)
    ltri = (ci < ri).astype(jnp.float32)
    parts = []
    running = jnp.zeros((1, E), jnp.float32)
    for b in range(2 * T // MT):
        blk = oh[b * MT:(b + 1) * MT]
        within = jnp.dot(ltri, blk, preferred_element_type=jnp.float32)
        parts.append(within + running)
        running = running + jnp.sum(blk, axis=0, keepdims=True)
    excl = jnp.concatenate(parts, axis=0)
    rank = jnp.sum(excl * oh, axis=-1)
    seg = jnp.ceil(running / float(MT)) * float(MT)
    segm = jnp.broadcast_to(seg, (E, E))
    je = jax.lax.broadcasted_iota(jnp.int32, (E, E), 1)
    ie = jax.lax.broadcasted_iota(jnp.int32, (E, E), 0)
    base = jnp.sum(jnp.where(je < ie, segm, 0.0), axis=-1)
    base_slot = jnp.sum(oh * base[None, :], axis=-1)
    pos_ref[...] = (base_slot + rank).astype(jnp.int32).reshape(1, 2 * T)
    tile_start = jax.lax.broadcasted_iota(jnp.int32, (NT, E), 0) * MT
    base_i = jnp.broadcast_to(base[None, :], (NT, E)).astype(jnp.int32)
    te_ref[...] = jnp.sum((base_i <= tile_start).astype(jnp.int32),
                          axis=-1, keepdims=True) - 1


def _gmm_kernel(te_ref, xl_ref, xr_ref, wg_ref, wu_ref, wd_ref, yl_ref, yr_ref):
    e = te_ref[pl.program_id(0), 0]
    xl = xl_ref[...]
    xr = xr_ref[...]
    wge = wg_ref[e]
    wue = wu_ref[e]
    wde = wd_ref[e]
    g = (jnp.dot(xl, wge[:HH], preferred_element_type=jnp.float32)
         + jnp.dot(xr, wge[HH:], preferred_element_type=jnp.float32))
    u = (jnp.dot(xl, wue[:HH], preferred_element_type=jnp.float32)
         + jnp.dot(xr, wue[HH:], preferred_element_type=jnp.float32))
    h = _silu(g) * u
    yl_ref[...] = jnp.dot(h, wde[:, :HH], preferred_element_type=jnp.float32)
    yr_ref[...] = jnp.dot(h, wde[:, HH:], preferred_element_type=jnp.float32)


def _shared_kernel(x_ref, sg_ref, su_ref, sd_ref, o_ref):
    x = x_ref[...]
    gs = jnp.dot(x, sg_ref[...], preferred_element_type=jnp.float32)
    us = jnp.dot(x, su_ref[...], preferred_element_type=jnp.float32)
    o_ref[...] = jnp.dot(_silu(gs) * us, sd_ref[...], preferred_element_type=jnp.float32)


def _combine_kernel(sh_ref, gl_ref, gr_ref, w1_ref, w2_ref, o_ref):
    w1 = w1_ref[...]
    w2 = w2_ref[...]
    o_ref[:, :HH] = (sh_ref[:, :HH]
                     + w1 * gl_ref[0:T, :] + w2 * gl_ref[T:2 * T, :])
    o_ref[:, HH:] = (sh_ref[:, HH:]
                     + w1 * gr_ref[0:T, :] + w2 * gr_ref[T:2 * T, :])


def _vec_mesh():
    return plsc.VectorSubcoreMesh(core_axis_name="core", subcore_axis_name="subcore")


def _sc_scatter(xf_hbm, pos_hbm, xsl_hbm, xsr_hbm):
    def body_l(x_vmem, i_vmem):
        pltpu.sync_copy(x_vmem, xsl_hbm.at[i_vmem.at[0]])

    def body_r(x_vmem, i_vmem):
        pltpu.sync_copy(x_vmem, xsr_hbm.at[i_vmem.at[0]])

    for body, col in ((body_l, 0), (body_r, 1)):
        pltpu.emit_pipeline(
            body,
            grid=(2 * T // W,),
            in_specs=[
                pl.BlockSpec((W, HH), lambda i, c=col: (i % (T // W), c)),
                pl.BlockSpec((1, W), lambda i: (0, i)),
            ],
            out_specs=[],
            core_axis_name=("core", "subcore"),
            dimension_semantics=(pltpu.PARALLEL,),
        )(xf_hbm, pos_hbm)


def _sc_gather(ysl_hbm, ysr_hbm, pos_hbm, gl_hbm, gr_hbm):
    def body_l(i_vmem, o_vmem):
        pltpu.sync_copy(ysl_hbm.at[i_vmem.at[0]], o_vmem)

    def body_r(i_vmem, o_vmem):
        pltpu.sync_copy(ysr_hbm.at[i_vmem.at[0]], o_vmem)

    for body, out in ((body_l, gl_hbm), (body_r, gr_hbm)):
        pltpu.emit_pipeline(
            body,
            grid=(2 * T // W,),
            in_specs=[pl.BlockSpec((1, W), lambda i: (0, i))],
            out_specs=[pl.BlockSpec((W, HH), lambda i: (i, 0))],
            core_axis_name=("core", "subcore"),
            dimension_semantics=(pltpu.PARALLEL,),
        )(pos_hbm, out)


def kernel(x, w_gate, wg, wu, wd, sg, su, sd):
    xf = x.reshape(T, H)

    pos, te, w1, w2 = pl.pallas_call(
        _router_kernel,
        out_shape=(
            jax.ShapeDtypeStruct((1, 2 * T), jnp.int32),
            jax.ShapeDtypeStruct((NT, 1), jnp.int32),
            jax.ShapeDtypeStruct((T, 1), jnp.float32),
            jax.ShapeDtypeStruct((T, 1), jnp.float32),
        ),
    )(xf, w_gate)

    sc_scatter = pl.kernel(
        _sc_scatter,
        out_type=(jax.ShapeDtypeStruct((M_PAD, HH), jnp.float32),
                  jax.ShapeDtypeStruct((M_PAD, HH), jnp.float32)),
        mesh=_vec_mesh(),
    )
    xsl, xsr = sc_scatter(xf, pos)

    ysl, ysr = pl.pallas_call(
        _gmm_kernel,
        grid=(NT,),
        in_specs=[
            pl.BlockSpec(memory_space=pltpu.SMEM),
            pl.BlockSpec((MT, HH), lambda i: (i, 0)),
            pl.BlockSpec((MT, HH), lambda i: (i, 0)),
            pl.BlockSpec((E, H, I), lambda i: (0, 0, 0)),
            pl.BlockSpec((E, H, I), lambda i: (0, 0, 0)),
            pl.BlockSpec((E, I, H), lambda i: (0, 0, 0)),
        ],
        out_specs=(pl.BlockSpec((MT, HH), lambda i: (i, 0)),
                   pl.BlockSpec((MT, HH), lambda i: (i, 0))),
        out_shape=(jax.ShapeDtypeStruct((M_PAD, HH), jnp.float32),
                   jax.ShapeDtypeStruct((M_PAD, HH), jnp.float32)),
        compiler_params=pltpu.CompilerParams(dimension_semantics=("arbitrary",)),
    )(te, xsl, xsr, wg, wu, wd)

    sc_gather = pl.kernel(
        _sc_gather,
        out_type=(jax.ShapeDtypeStruct((2 * T, HH), jnp.float32),
                  jax.ShapeDtypeStruct((2 * T, HH), jnp.float32)),
        mesh=_vec_mesh(),
    )
    gl, gr = sc_gather(ysl, ysr, pos)

    shared = pl.pallas_call(
        _shared_kernel,
        out_shape=jax.ShapeDtypeStruct((T, H), jnp.float32),
    )(xf, sg, su, sd)

    y = pl.pallas_call(
        _combine_kernel,
        out_shape=jax.ShapeDtypeStruct((T, H), jnp.float32),
    )(shared, gl, gr, w1, w2)

    return y.reshape(B, S, H)


# speedup vs baseline: 2.5135x; 1.5747x over previous
import jax
import jax.numpy as jnp
from jax.experimental import pallas as pl
from jax.experimental.pallas import tpu as pltpu

B, S, H = 1, 2048, 768
E, K, I = 16, 2, 256
T = B * S


def _router_kernel(x_ref, wg_ref, comb_ref):
    logits = jnp.dot(x_ref[...], wg_ref[...], preferred_element_type=jnp.float32)
    iota_e = jax.lax.broadcasted_iota(jnp.int32, logits.shape, 1)
    a1 = jnp.argmax(logits, axis=-1)
    hot1 = iota_e == a1[:, None]
    m1 = jnp.max(logits, axis=-1, keepdims=True)
    masked = jnp.where(hot1, -jnp.inf, logits)
    a2 = jnp.argmax(masked, axis=-1)
    hot2 = iota_e == a2[:, None]
    m2 = jnp.max(masked, axis=-1, keepdims=True)
    e2 = jnp.exp(m2 - m1)
    w1 = 1.0 / (1.0 + e2)
    w2 = e2 / (1.0 + e2)
    comb = jnp.where(hot1, w1, 0.0) + jnp.where(hot2, w2, 0.0)
    comb_ref[...] = comb.T.reshape(E, 1, T)


def _silu(v):
    return v * jax.nn.sigmoid(v)


def _moe_kernel(comb_ref, x_ref, wg_ref, wu_ref, wd_ref, sg_ref, su_ref, sd_ref,
                o_ref):
    e = pl.program_id(0)
    x = x_ref[...]
    g = jnp.dot(x, wg_ref[0], preferred_element_type=jnp.float32)
    u = jnp.dot(x, wu_ref[0], preferred_element_type=jnp.float32)
    h = _silu(g) * u * comb_ref[0, 0].reshape(T, 1)
    y = jnp.dot(h, wd_ref[0], preferred_element_type=jnp.float32)

    @pl.when(e == 0)
    def _():
        gs = jnp.dot(x, sg_ref[...], preferred_element_type=jnp.float32)
        us = jnp.dot(x, su_ref[...], preferred_element_type=jnp.float32)
        hs = _silu(gs) * us
        o_ref[...] = y + jnp.dot(hs, sd_ref[...], preferred_element_type=jnp.float32)

    @pl.when(e != 0)
    def _():
        o_ref[...] += y


def kernel(x, w_gate, wg, wu, wd, sg, su, sd):
    xf = x.reshape(T, H)

    comb = pl.pallas_call(
        _router_kernel,
        out_shape=jax.ShapeDtypeStruct((E, 1, T), jnp.float32),
    )(xf, w_gate)

    y = pl.pallas_call(
        _moe_kernel,
        grid=(E,),
        in_specs=[
            pl.BlockSpec((1, 1, T), lambda e: (e, 0, 0)),
            pl.BlockSpec((T, H), lambda e: (0, 0)),
            pl.BlockSpec((1, H, I), lambda e: (e, 0, 0)),
            pl.BlockSpec((1, H, I), lambda e: (e, 0, 0)),
            pl.BlockSpec((1, I, H), lambda e: (e, 0, 0)),
            pl.BlockSpec((H, I), lambda e: (0, 0)),
            pl.BlockSpec((H, I), lambda e: (0, 0)),
            pl.BlockSpec((I, H), lambda e: (0, 0)),
        ],
        out_specs=pl.BlockSpec((T, H), lambda e: (0, 0)),
        out_shape=jax.ShapeDtypeStruct((T, H), jnp.float32),
        compiler_params=pltpu.CompilerParams(
            dimension_semantics=("arbitrary",),
        ),
    )(comb, xf, wg, wu, wd, sg, su, sd)

    return y.reshape(B, S, H)
